# Initial kernel scaffold; baseline (speedup 1.0000x reference)
import jax
import jax.numpy as jnp
from jax.experimental import pallas as pl


def kernel(node_feat, edge_attr, edge_index, batch, W1n, b1n, W2n, b2n, W1e, b1e, W2e, b2e, theta, Wu1, bu1, Wu2, bu2, gamma, beta, Wg1, bg1, Wg2, bg2):
    raise NotImplementedError("write your pallas kernel here")


# speedup vs baseline: 1.0732x; 1.0732x over previous
import functools

import jax
import jax.numpy as jnp
import numpy as np
from jax.experimental import pallas as pl
from jax.experimental.pallas import tpu as pltpu

N = 10000; E = 320000; DF = 128; DE = 4; H = 128; P = 2; K = 4; NG = 64; NC = 2
BN = 1000
NB = N // BN
_PI = np.float32(np.pi)


def _leaky(x):
    return jnp.where(x > 0, x, 0.2 * x)


def _node_mlp_body(x_ref, w1_ref, b1_ref, w2_ref, b2_ref, o_ref):
    h = jnp.dot(x_ref[...], w1_ref[...], preferred_element_type=jnp.float32)
    h = h + b1_ref[...]
    h = _leaky(h)
    o = jnp.dot(h, w2_ref[...], preferred_element_type=jnp.float32) + b2_ref[...]
    o_ref[...] = jnp.tanh(o) * _PI


def _node_mlp(node_feat, W1n, b1n, W2n, b2n):
    return pl.pallas_call(
        _node_mlp_body,
        grid=(NB,),
        in_specs=[
            pl.BlockSpec((BN, DF), lambda b: (b, 0)),
            pl.BlockSpec((DF, H), lambda b: (0, 0)),
            pl.BlockSpec((1, H), lambda b: (0, 0)),
            pl.BlockSpec((H, P), lambda b: (0, 0)),
            pl.BlockSpec((1, P), lambda b: (0, 0)),
        ],
        out_specs=pl.BlockSpec((BN, P), lambda b: (b, 0)),
        out_shape=jax.ShapeDtypeStruct((N, P), jnp.float32),
    )(node_feat, W1n, b1n.reshape(1, H), W2n, b2n.reshape(1, P))


def _tail_body(nf_ref, n1_ref, n2_ref, n3_ref, ea1_ref, ea2_ref, ea3_ref,
               cnt_ref, bat_ref, theta_ref,
               w1e_ref, b1e_ref, w2e_ref, b2e_ref,
               wu1_ref, bu1_ref, wu2_ref, bu2_ref,
               gam_ref, bet_ref, wg1_ref, bg1_ref, wg2_ref, bg2_ref,
               o_ref, gsum_s, gcnt_s):
    b = pl.program_id(0)

    @pl.when(b == 0)
    def _init():
        gsum_s[...] = jnp.zeros_like(gsum_s)
        gcnt_s[...] = jnp.zeros_like(gcnt_s)

    nf = nf_ref[...]
    cnt = cnt_ref[0]
    w1e = w1e_ref[...]; b1e = b1e_ref[...]
    w2e = w2e_ref[...]; b2e = b2e_ref[...]

    def edge_mlp(ea, j):
        h = jnp.dot(ea, w1e, preferred_element_type=jnp.float32) + b1e
        h = _leaky(h)
        ef = jnp.tanh(jnp.dot(h, w2e, preferred_element_type=jnp.float32) + b2e) * _PI
        return jnp.where(cnt > j, ef, 0.0)

    ef1 = edge_mlp(ea1_ref[...], 0.0)
    ef2 = edge_mlp(ea2_ref[...], 1.0)
    ef3 = edge_mlp(ea3_ref[...], 2.0)
    n1 = jnp.where(cnt > 0.0, n1_ref[...], 0.0)
    n2 = jnp.where(cnt > 1.0, n2_ref[...], 0.0)
    n3 = jnp.where(cnt > 2.0, n3_ref[...], 0.0)

    cth = jnp.cos(0.5 * jnp.sum(theta_ref[...]))

    def cprod(x):
        c = jnp.cos(0.5 * x)
        return c[:, 0:1] * c[:, 1:2]

    p_nf = cprod(nf); p_n1 = cprod(n1)
    pqc = (p_nf * p_nf) * (p_n1 * p_n1) * cprod(n2) * cprod(n3)
    pqc = pqc * cprod(ef1) * cprod(ef2) * cprod(ef3) * cth

    deg = jnp.maximum(jnp.minimum(cnt, np.float32(K - 1)), 1.0)
    neigh_mean = (n1 + n2 + n3) / deg

    wu1 = wu1_ref[...]
    upre = (jnp.dot(nf, wu1[0:P, :], preferred_element_type=jnp.float32)
            + pqc * wu1[P:P + 1, :]
            + jnp.dot(neigh_mean, wu1[P + 1:2 * P + 1, :],
                      preferred_element_type=jnp.float32)
            + bu1_ref[...])
    u = jnp.dot(_leaky(upre), wu2_ref[...], preferred_element_type=jnp.float32)
    u = u + bu2_ref[...]

    mu = jnp.mean(u, axis=-1, keepdims=True)
    d = u - mu
    var = jnp.mean(d * d, axis=-1, keepdims=True)
    un = d / jnp.sqrt(var + 1e-3) * gam_ref[...] + bet_ref[...]

    bat = bat_ref[0]
    seg = jnp.arange(NG, dtype=jnp.int32).reshape(1, NG)
    onehot = (bat == seg).astype(jnp.float32)
    gsum_s[...] += jax.lax.dot_general(
        onehot, un, (((0,), (0,)), ((), ())), preferred_element_type=jnp.float32)
    gcnt_s[...] += jnp.sum(onehot, axis=0, keepdims=True)

    @pl.when(b == NB - 1)
    def _fin():
        gmean = gsum_s[...] / jnp.maximum(gcnt_s[...].reshape(NG, 1), 1.0)
        g = _leaky(jnp.dot(gmean, wg1_ref[...], preferred_element_type=jnp.float32)
                   + bg1_ref[...])
        o_ref[...] = (jnp.dot(g, wg2_ref[...], preferred_element_type=jnp.float32)
                      + bg2_ref[...])


def _tail(nf, n1, n2, n3, ea1, ea2, ea3, cntf, bat, theta,
          W1e, b1e, W2e, b2e, Wu1, bu1, Wu2, bu2, gamma, beta,
          Wg1, bg1, Wg2, bg2):
    node2 = pl.BlockSpec((BN, P), lambda b: (b, 0))
    node4 = pl.BlockSpec((BN, DE), lambda b: (b, 0))
    col = pl.BlockSpec((1, BN, 1), lambda b: (b, 0, 0))
    full = lambda s: pl.BlockSpec(s, lambda b: (0, 0))
    return pl.pallas_call(
        _tail_body,
        grid=(NB,),
        in_specs=[node2, node2, node2, node2, node4, node4, node4,
                  col, col, full((1, 27)),
                  full((DE, H)), full((1, H)), full((H, P)), full((1, P)),
                  full((2 * P + 1, H)), full((1, H)), full((H, P)), full((1, P)),
                  full((1, P)), full((1, P)),
                  full((P, NC)), full((1, NC)), full((NC, NC)), full((1, NC))],
        out_specs=pl.BlockSpec((NG, NC), lambda b: (0, 0)),
        out_shape=jax.ShapeDtypeStruct((NG, NC), jnp.float32),
        scratch_shapes=[pltpu.VMEM((NG, P), jnp.float32),
                        pltpu.VMEM((1, NG), jnp.float32)],
    )(nf, n1, n2, n3, ea1, ea2, ea3, cntf, bat, theta.reshape(1, 27),
      W1e, b1e.reshape(1, H), W2e, b2e.reshape(1, P),
      Wu1, bu1.reshape(1, H), Wu2, bu2.reshape(1, P),
      gamma.reshape(1, P), beta.reshape(1, P),
      Wg1, bg1.reshape(1, NC), Wg2, bg2.reshape(1, NC))


def kernel(node_feat, edge_attr, edge_index, batch, W1n, b1n, W2n, b2n,
           W1e, b1e, W2e, b2e, theta, Wu1, bu1, Wu2, bu2, gamma, beta,
           Wg1, bg1, Wg2, bg2):
    nf = _node_mlp(node_feat, W1n, b1n, W2n, b2n)

    src = edge_index[0]
    dst = edge_index[1]
    order = jnp.argsort(src)
    counts = jnp.bincount(src, length=N)
    offsets = jnp.concatenate(
        [jnp.zeros((1,), counts.dtype), jnp.cumsum(counts)])[:-1]
    nfs = []; eas = []
    for j in range(K - 1):
        posc = jnp.clip(offsets + j, 0, E - 1)
        eid = order[posc]
        nfs.append(nf[dst[eid]])
        eas.append(edge_attr[eid])

    cntf = counts.astype(jnp.float32).reshape(NB, BN, 1)
    bat = batch.reshape(NB, BN, 1)
    return _tail(nf, nfs[0], nfs[1], nfs[2], eas[0], eas[1], eas[2],
                 cntf, bat, theta, W1e, b1e, W2e, b2e, Wu1, bu1, Wu2, bu2,
                 gamma, beta, Wg1, bg1, Wg2, bg2)


# speedup vs baseline: 2.4840x; 2.3145x over previous
import functools

import jax
import jax.numpy as jnp
import numpy as np
from jax import lax
from jax.experimental import pallas as pl
from jax.experimental.pallas import tpu as pltpu
from jax.experimental.pallas import tpu_sc as plsc

N = 10000; E = 320000; DF = 128; DE = 4; H = 128; P = 2; K = 4; NG = 64; NC = 2
BN = 1000
NB = N // BN
_PI = np.float32(np.pi)

NW = 32
EC = E // NW
NPW = 320
NPAD = NW * NPW
CH = 80
NCH = NPW // CH


def _leaky(x):
    return jnp.where(x > 0, x, 0.2 * x)


def _node_mlp_body(x_ref, w1_ref, b1_ref, w2_ref, b2_ref, o_ref):
    h = jnp.dot(x_ref[...], w1_ref[...], preferred_element_type=jnp.float32)
    h = h + b1_ref[...]
    h = _leaky(h)
    o = jnp.dot(h, w2_ref[...], preferred_element_type=jnp.float32) + b2_ref[...]
    o_ref[...] = jnp.tanh(o) * _PI


def _node_mlp(node_feat, W1n, b1n, W2n, b2n):
    return pl.pallas_call(
        _node_mlp_body,
        grid=(NB,),
        in_specs=[
            pl.BlockSpec((BN, DF), lambda b: (b, 0)),
            pl.BlockSpec((DF, H), lambda b: (0, 0)),
            pl.BlockSpec((1, H), lambda b: (0, 0)),
            pl.BlockSpec((H, P), lambda b: (0, 0)),
            pl.BlockSpec((1, P), lambda b: (0, 0)),
        ],
        out_specs=pl.BlockSpec((BN, P), lambda b: (b, 0)),
        out_shape=jax.ShapeDtypeStruct((N, P), jnp.float32),
    )(node_feat, W1n, b1n.reshape(1, H), W2n, b2n.reshape(1, P))


def _sc_select_body(src_hbm, lcnt_hbm, l3_hbm, src_v, lcnt_v, l3_v):
    wid = lax.axis_index("c") * 16 + lax.axis_index("s")
    ebase = wid * EC
    pltpu.sync_copy(src_hbm.at[pl.ds(ebase, EC)], src_v)

    @pl.loop(0, NPAD // 16)
    def _zero(i):
        lcnt_v[pl.ds(i * 16, 16)] = jnp.zeros((16,), jnp.int32)

    v0raw, _ = plsc.scan_count(jnp.zeros((16,), jnp.int32))
    v0 = v0raw - lax.iota(jnp.int32, 16)

    @pl.loop(0, EC // 16)
    def _scan(i):
        s = src_v[pl.ds(i * 16, 16)]
        raw, last = plsc.scan_count(s)
        pd = raw - v0
        old = plsc.load_gather(lcnt_v, [s])
        rank = old + pd
        eid = ebase + i * 16 + lax.iota(jnp.int32, 16)
        sel = rank < 3
        idx = s * 3 + jnp.where(sel, rank, 0)
        plsc.store_scatter(l3_v, [idx], eid, mask=sel)
        plsc.store_scatter(lcnt_v, [s], rank + 1, mask=last)

    pltpu.sync_copy(lcnt_v, lcnt_hbm.at[pl.ds(wid * NPAD, NPAD)])
    pltpu.sync_copy(l3_v, l3_hbm.at[pl.ds(wid * 3 * NPAD, 3 * NPAD)])


def _sc_select(src):
    mesh = plsc.VectorSubcoreMesh(core_axis_name="c", subcore_axis_name="s")
    return pl.kernel(
        _sc_select_body,
        out_type=(jax.ShapeDtypeStruct((NW * NPAD,), jnp.int32),
                  jax.ShapeDtypeStruct((NW * 3 * NPAD,), jnp.int32)),
        mesh=mesh,
        compiler_params=pltpu.CompilerParams(needs_layout_passes=False),
        scratch_types=[pltpu.VMEM((EC,), jnp.int32),
                       pltpu.VMEM((NPAD,), jnp.int32),
                       pltpu.VMEM((3 * NPAD,), jnp.int32)],
    )(src)


def _sc_merge_body(lcnt_hbm, l3_hbm, dst_hbm, ea0_hbm, ea1_hbm, ea2_hbm,
                   ea3_hbm, nfc0_hbm, nfc1_hbm, *rest):
    outs = rest[:19]
    cnt_out = outs[0]
    nf_outs = outs[1:7]
    ea_outs = outs[7:19]
    lcnt_v, l3_v, e_v, cnt_v, nbr_v, g_v, sem = rest[19:]
    ea_hbms = (ea0_hbm, ea1_hbm, ea2_hbm, ea3_hbm)

    wid = lax.axis_index("c") * 16 + lax.axis_index("s")
    nbase = wid * NPW
    nreal = jnp.where(wid == NW - 1, N - (NW - 1) * NPW, NPW)

    descs = []
    for w in range(NW):
        descs.append(pltpu.async_copy(
            lcnt_hbm.at[pl.ds(w * NPAD + nbase, NPW)],
            lcnt_v.at[pl.ds(w * NPW, NPW)], sem))
        descs.append(pltpu.async_copy(
            l3_hbm.at[pl.ds(w * 3 * NPAD + 3 * nbase, 3 * NPW)],
            l3_v.at[pl.ds(w * 3 * NPW, 3 * NPW)], sem))
    for d in descs:
        d.wait()

    @pl.loop(0, nreal // 16)
    def _merge(k):
        b = jnp.zeros((16,), jnp.int32)
        e = [jnp.zeros((16,), jnp.int32) for _ in range(3)]
        base_idx = (k * 16 + lax.iota(jnp.int32, 16)) * 3
        for w in range(NW):
            cw = lcnt_v[pl.ds(w * NPW + k * 16, 16)]
            for r in range(3):
                sel = (cw > r) & (b + r < 3)
                eid = plsc.load_gather(
                    l3_v, [w * 3 * NPW + base_idx + r], mask=sel)
                gr = b + r
                for j in range(3):
                    e[j] = jnp.where(sel & (gr == j), eid, e[j])
            b = b + cw
        cnt_v[pl.ds(k * 16, 16)] = jnp.minimum(b, 3).astype(jnp.float32)
        for j in range(3):
            e_v[pl.ds(j * NPW + k * 16, 16)] = e[j]

    @pl.loop(0, nreal // CH)
    def _gather(c):
        ob = nbase + c * CH
        ds1 = [pltpu.async_copy(dst_hbm.at[e_v.at[pl.ds(j * NPW + c * CH, CH)]],
                                nbr_v.at[pl.ds(j * CH, CH)], sem)
               for j in range(3)]
        for j in range(3):
            for dcol in range(4):
                ds1.append(pltpu.async_copy(
                    ea_hbms[dcol].at[e_v.at[pl.ds(j * NPW + c * CH, CH)]],
                    g_v.at[pl.ds((j * 4 + dcol) * CH, CH)], sem))
        for d in ds1:
            d.wait()
        ds2 = [pltpu.async_copy(
                   (nfc0_hbm, nfc1_hbm)[comp].at[nbr_v.at[pl.ds(j * CH, CH)]],
                   g_v.at[pl.ds((12 + j * 2 + comp) * CH, CH)], sem)
               for j in range(3) for comp in range(2)]
        ds2 += [pltpu.async_copy(g_v.at[pl.ds((j * 4 + dcol) * CH, CH)],
                                 ea_outs[j * 4 + dcol].at[pl.ds(ob, CH)], sem)
                for j in range(3) for dcol in range(4)]
        for d in ds2:
            d.wait()
        ds3 = [pltpu.async_copy(g_v.at[pl.ds((12 + j * 2 + comp) * CH, CH)],
                                nf_outs[j * 2 + comp].at[pl.ds(ob, CH)], sem)
               for j in range(3) for comp in range(2)]
        ds3.append(pltpu.async_copy(cnt_v.at[pl.ds(c * CH, CH)],
                                    cnt_out.at[pl.ds(ob, CH)], sem))
        for d in ds3:
            d.wait()


def _sc_merge(lcnt_all, l3_all, dst, ea_cols, nf_cols):
    mesh = plsc.VectorSubcoreMesh(core_axis_name="c", subcore_axis_name="s")
    outs = pl.kernel(
        _sc_merge_body,
        out_type=tuple(jax.ShapeDtypeStruct((N,), jnp.float32)
                       for _ in range(19)),
        mesh=mesh,
        compiler_params=pltpu.CompilerParams(needs_layout_passes=False),
        scratch_types=[pltpu.VMEM((NW * NPW,), jnp.int32),
                       pltpu.VMEM((NW * 3 * NPW,), jnp.int32),
                       pltpu.VMEM((3 * NPW,), jnp.int32),
                       pltpu.VMEM((NPW,), jnp.float32),
                       pltpu.VMEM((3 * CH,), jnp.int32),
                       pltpu.VMEM((18 * CH,), jnp.float32),
                       pltpu.SemaphoreType.DMA],
    )(lcnt_all, l3_all, dst, *ea_cols, *nf_cols)
    return outs


def _tail_body(nf_ref, n1_ref, n2_ref, n3_ref, ea1_ref, ea2_ref, ea3_ref,
               cnt_ref, bat_ref, theta_ref,
               w1e_ref, b1e_ref, w2e_ref, b2e_ref,
               wu1_ref, bu1_ref, wu2_ref, bu2_ref,
               gam_ref, bet_ref, wg1_ref, bg1_ref, wg2_ref, bg2_ref,
               o_ref, gsum_s, gcnt_s):
    b = pl.program_id(0)

    @pl.when(b == 0)
    def _init():
        gsum_s[...] = jnp.zeros_like(gsum_s)
        gcnt_s[...] = jnp.zeros_like(gcnt_s)

    nf = nf_ref[...]
    cnt = cnt_ref[0]
    w1e = w1e_ref[...]; b1e = b1e_ref[...]
    w2e = w2e_ref[...]; b2e = b2e_ref[...]

    def edge_mlp(ea, j):
        h = jnp.dot(ea, w1e, preferred_element_type=jnp.float32) + b1e
        h = _leaky(h)
        ef = jnp.tanh(jnp.dot(h, w2e, preferred_element_type=jnp.float32) + b2e) * _PI
        return jnp.where(cnt > j, ef, 0.0)

    ef1 = edge_mlp(ea1_ref[...], 0.0)
    ef2 = edge_mlp(ea2_ref[...], 1.0)
    ef3 = edge_mlp(ea3_ref[...], 2.0)
    n1 = jnp.where(cnt > 0.0, n1_ref[...], 0.0)
    n2 = jnp.where(cnt > 1.0, n2_ref[...], 0.0)
    n3 = jnp.where(cnt > 2.0, n3_ref[...], 0.0)

    cth = jnp.cos(0.5 * jnp.sum(theta_ref[...]))

    def cprod(x):
        c = jnp.cos(0.5 * x)
        return c[:, 0:1] * c[:, 1:2]

    p_nf = cprod(nf); p_n1 = cprod(n1)
    pqc = (p_nf * p_nf) * (p_n1 * p_n1) * cprod(n2) * cprod(n3)
    pqc = pqc * cprod(ef1) * cprod(ef2) * cprod(ef3) * cth

    deg = jnp.maximum(jnp.minimum(cnt, np.float32(K - 1)), 1.0)
    neigh_mean = (n1 + n2 + n3) / deg

    wu1 = wu1_ref[...]
    upre = (jnp.dot(nf, wu1[0:P, :], preferred_element_type=jnp.float32)
            + pqc * wu1[P:P + 1, :]
            + jnp.dot(neigh_mean, wu1[P + 1:2 * P + 1, :],
                      preferred_element_type=jnp.float32)
            + bu1_ref[...])
    u = jnp.dot(_leaky(upre), wu2_ref[...], preferred_element_type=jnp.float32)
    u = u + bu2_ref[...]

    mu = jnp.mean(u, axis=-1, keepdims=True)
    d = u - mu
    var = jnp.mean(d * d, axis=-1, keepdims=True)
    un = d / jnp.sqrt(var + 1e-3) * gam_ref[...] + bet_ref[...]

    bat = bat_ref[0]
    seg = jnp.arange(NG, dtype=jnp.int32).reshape(1, NG)
    onehot = (bat == seg).astype(jnp.float32)
    gsum_s[...] += jax.lax.dot_general(
        onehot, un, (((0,), (0,)), ((), ())), preferred_element_type=jnp.float32)
    gcnt_s[...] += jnp.sum(onehot, axis=0, keepdims=True)

    @pl.when(b == NB - 1)
    def _fin():
        gmean = gsum_s[...] / jnp.maximum(gcnt_s[...].reshape(NG, 1), 1.0)
        g = _leaky(jnp.dot(gmean, wg1_ref[...], preferred_element_type=jnp.float32)
                   + bg1_ref[...])
        o_ref[...] = (jnp.dot(g, wg2_ref[...], preferred_element_type=jnp.float32)
                      + bg2_ref[...])


def _tail(nf, n1, n2, n3, ea1, ea2, ea3, cntf, bat, theta,
          W1e, b1e, W2e, b2e, Wu1, bu1, Wu2, bu2, gamma, beta,
          Wg1, bg1, Wg2, bg2):
    node2 = pl.BlockSpec((BN, P), lambda b: (b, 0))
    node4 = pl.BlockSpec((BN, DE), lambda b: (b, 0))
    col = pl.BlockSpec((1, BN, 1), lambda b: (b, 0, 0))
    full = lambda s: pl.BlockSpec(s, lambda b: (0, 0))
    return pl.pallas_call(
        _tail_body,
        grid=(NB,),
        in_specs=[node2, node2, node2, node2, node4, node4, node4,
                  col, col, full((1, 27)),
                  full((DE, H)), full((1, H)), full((H, P)), full((1, P)),
                  full((2 * P + 1, H)), full((1, H)), full((H, P)), full((1, P)),
                  full((1, P)), full((1, P)),
                  full((P, NC)), full((1, NC)), full((NC, NC)), full((1, NC))],
        out_specs=pl.BlockSpec((NG, NC), lambda b: (0, 0)),
        out_shape=jax.ShapeDtypeStruct((NG, NC), jnp.float32),
        scratch_shapes=[pltpu.VMEM((NG, P), jnp.float32),
                        pltpu.VMEM((1, NG), jnp.float32)],
    )(nf, n1, n2, n3, ea1, ea2, ea3, cntf, bat, theta.reshape(1, 27),
      W1e, b1e.reshape(1, H), W2e, b2e.reshape(1, P),
      Wu1, bu1.reshape(1, H), Wu2, bu2.reshape(1, P),
      gamma.reshape(1, P), beta.reshape(1, P),
      Wg1, bg1.reshape(1, NC), Wg2, bg2.reshape(1, NC))


def kernel(node_feat, edge_attr, edge_index, batch, W1n, b1n, W2n, b2n,
           W1e, b1e, W2e, b2e, theta, Wu1, bu1, Wu2, bu2, gamma, beta,
           Wg1, bg1, Wg2, bg2):
    nf = _node_mlp(node_feat, W1n, b1n, W2n, b2n)

    src = edge_index[0]
    dst = edge_index[1]
    lcnt_all, l3_all = _sc_select(src)
    ea_cols = [edge_attr[:, d] for d in range(DE)]
    nf_cols = [nf[:, c] for c in range(P)]
    outs = _sc_merge(lcnt_all, l3_all, dst, ea_cols, nf_cols)
    cnt3 = outs[0]
    nf1, nf2, nf3 = (jnp.stack([outs[1 + 2 * j], outs[2 + 2 * j]], axis=1)
                     for j in range(3))
    ea1, ea2, ea3 = (jnp.stack([outs[7 + 4 * j + d] for d in range(4)], axis=1)
                     for j in range(3))

    cntf = cnt3.reshape(NB, BN, 1)
    bat = batch.reshape(NB, BN, 1)
    return _tail(nf, nf1, nf2, nf3, ea1, ea2, ea3,
                 cntf, bat, theta, W1e, b1e, W2e, b2e, Wu1, bu1, Wu2, bu2,
                 gamma, beta, Wg1, bg1, Wg2, bg2)


# speedup vs baseline: 5.0739x; 2.0427x over previous
import functools

import jax
import jax.numpy as jnp
import numpy as np
from jax import lax
from jax.experimental import pallas as pl
from jax.experimental.pallas import tpu as pltpu
from jax.experimental.pallas import tpu_sc as plsc

N = 10000; E = 320000; DF = 128; DE = 4; H = 128; P = 2; K = 4; NG = 64; NC = 2
BN = 1000
NB = N // BN
_PI = np.float32(np.pi)

NW = 32
EC = E // NW
NPW = 320
NPAD = NW * NPW
CH = 80
NCH = NPW // CH


def _leaky(x):
    return jnp.where(x > 0, x, 0.2 * x)


def _node_mlp_body(x_ref, w1_ref, b1_ref, w2_ref, b2_ref, o_ref):
    h = jnp.dot(x_ref[...], w1_ref[...], preferred_element_type=jnp.float32)
    h = h + b1_ref[...]
    h = _leaky(h)
    o = jnp.dot(h, w2_ref[...], preferred_element_type=jnp.float32) + b2_ref[...]
    o_ref[...] = jnp.tanh(o) * _PI


def _node_mlp(node_feat, W1n, b1n, W2n, b2n):
    return pl.pallas_call(
        _node_mlp_body,
        grid=(NB,),
        in_specs=[
            pl.BlockSpec((BN, DF), lambda b: (b, 0)),
            pl.BlockSpec((DF, H), lambda b: (0, 0)),
            pl.BlockSpec((1, H), lambda b: (0, 0)),
            pl.BlockSpec((H, P), lambda b: (0, 0)),
            pl.BlockSpec((1, P), lambda b: (0, 0)),
        ],
        out_specs=pl.BlockSpec((BN, P), lambda b: (b, 0)),
        out_shape=jax.ShapeDtypeStruct((N, P), jnp.float32),
    )(node_feat, W1n, b1n.reshape(1, H), W2n, b2n.reshape(1, P))


def _sc_select_body(src_hbm, lcnt_hbm, l3_hbm, src_v, lcnt_v, l3_v):
    wid = lax.axis_index("c") * 16 + lax.axis_index("s")
    ebase = wid * EC
    pltpu.sync_copy(src_hbm.at[pl.ds(ebase, EC)], src_v)

    @pl.loop(0, NPAD // 16)
    def _zero(i):
        lcnt_v[pl.ds(i * 16, 16)] = jnp.zeros((16,), jnp.int32)

    v0raw, _ = plsc.scan_count(jnp.zeros((16,), jnp.int32))
    v0 = v0raw - lax.iota(jnp.int32, 16)

    @pl.loop(0, EC // 16)
    def _scan(i):
        s = src_v[pl.ds(i * 16, 16)]
        raw, last = plsc.scan_count(s)
        pd = raw - v0
        old = plsc.load_gather(lcnt_v, [s])
        rank = old + pd
        eid = ebase + i * 16 + lax.iota(jnp.int32, 16)
        sel = rank < 3
        idx = s * 3 + jnp.where(sel, rank, 0)
        plsc.store_scatter(l3_v, [idx], eid, mask=sel)
        plsc.store_scatter(lcnt_v, [s], rank + 1, mask=last)

    pltpu.sync_copy(lcnt_v, lcnt_hbm.at[pl.ds(wid * NPAD, NPAD)])
    pltpu.sync_copy(l3_v, l3_hbm.at[pl.ds(wid * 3 * NPAD, 3 * NPAD)])


def _sc_select(src):
    mesh = plsc.VectorSubcoreMesh(core_axis_name="c", subcore_axis_name="s")
    return pl.kernel(
        _sc_select_body,
        out_type=(jax.ShapeDtypeStruct((NW * NPAD,), jnp.int32),
                  jax.ShapeDtypeStruct((NW * 3 * NPAD,), jnp.int32)),
        mesh=mesh,
        compiler_params=pltpu.CompilerParams(needs_layout_passes=False),
        scratch_types=[pltpu.VMEM((EC,), jnp.int32),
                       pltpu.VMEM((NPAD,), jnp.int32),
                       pltpu.VMEM((3 * NPAD,), jnp.int32)],
    )(src)


def _sc_merge_body(lcnt_hbm, l3_hbm, dst_hbm, ea0_hbm, ea1_hbm, ea2_hbm,
                   ea3_hbm, nfc0_hbm, nfc1_hbm, *rest):
    outs = rest[:19]
    cnt_out = outs[0]
    nf_outs = outs[1:7]
    ea_outs = outs[7:19]
    lcnt_v, l3_v, e_v, cnt_v, nbr_v, g_v, sem = rest[19:]
    ea_hbms = (ea0_hbm, ea1_hbm, ea2_hbm, ea3_hbm)

    wid = lax.axis_index("c") * 16 + lax.axis_index("s")
    nbase = wid * NPW
    nreal = jnp.where(wid == NW - 1, N - (NW - 1) * NPW, NPW)

    descs = []
    for w in range(NW):
        descs.append(pltpu.async_copy(
            lcnt_hbm.at[pl.ds(w * NPAD + nbase, NPW)],
            lcnt_v.at[pl.ds(w * NPW, NPW)], sem))
        descs.append(pltpu.async_copy(
            l3_hbm.at[pl.ds(w * 3 * NPAD + 3 * nbase, 3 * NPW)],
            l3_v.at[pl.ds(w * 3 * NPW, 3 * NPW)], sem))
    for d in descs:
        d.wait()

    @pl.loop(0, nreal // 16)
    def _merge(k):
        b = jnp.zeros((16,), jnp.int32)
        e = [jnp.zeros((16,), jnp.int32) for _ in range(3)]
        base_idx = (k * 16 + lax.iota(jnp.int32, 16)) * 3
        for w in range(NW):
            cw = lcnt_v[pl.ds(w * NPW + k * 16, 16)]
            for r in range(3):
                sel = (cw > r) & (b + r < 3)
                eid = plsc.load_gather(
                    l3_v, [w * 3 * NPW + base_idx + r], mask=sel)
                gr = b + r
                for j in range(3):
                    e[j] = jnp.where(sel & (gr == j), eid, e[j])
            b = b + cw
        cnt_v[pl.ds(k * 16, 16)] = jnp.minimum(b, 3).astype(jnp.float32)
        for j in range(3):
            e_v[pl.ds(j * NPW + k * 16, 16)] = e[j]

    @pl.loop(0, nreal // CH)
    def _gather(c):
        ob = nbase + c * CH
        ds1 = [pltpu.async_copy(dst_hbm.at[e_v.at[pl.ds(j * NPW + c * CH, CH)]],
                                nbr_v.at[pl.ds(j * CH, CH)], sem)
               for j in range(3)]
        for j in range(3):
            for dcol in range(4):
                ds1.append(pltpu.async_copy(
                    ea_hbms[dcol].at[e_v.at[pl.ds(j * NPW + c * CH, CH)]],
                    g_v.at[pl.ds((j * 4 + dcol) * CH, CH)], sem))
        for d in ds1:
            d.wait()
        ds2 = [pltpu.async_copy(
                   (nfc0_hbm, nfc1_hbm)[comp].at[nbr_v.at[pl.ds(j * CH, CH)]],
                   g_v.at[pl.ds((12 + j * 2 + comp) * CH, CH)], sem)
               for j in range(3) for comp in range(2)]
        ds2 += [pltpu.async_copy(g_v.at[pl.ds((j * 4 + dcol) * CH, CH)],
                                 ea_outs[j * 4 + dcol].at[pl.ds(ob, CH)], sem)
                for j in range(3) for dcol in range(4)]
        for d in ds2:
            d.wait()
        ds3 = [pltpu.async_copy(g_v.at[pl.ds((12 + j * 2 + comp) * CH, CH)],
                                nf_outs[j * 2 + comp].at[pl.ds(ob, CH)], sem)
               for j in range(3) for comp in range(2)]
        ds3.append(pltpu.async_copy(cnt_v.at[pl.ds(c * CH, CH)],
                                    cnt_out.at[pl.ds(ob, CH)], sem))
        for d in ds3:
            d.wait()


def _sc_merge(lcnt_all, l3_all, dst, ea_cols, nf_cols):
    mesh = plsc.VectorSubcoreMesh(core_axis_name="c", subcore_axis_name="s")
    outs = pl.kernel(
        _sc_merge_body,
        out_type=tuple(jax.ShapeDtypeStruct((N,), jnp.float32)
                       for _ in range(19)),
        mesh=mesh,
        compiler_params=pltpu.CompilerParams(needs_layout_passes=False),
        scratch_types=[pltpu.VMEM((NW * NPW,), jnp.int32),
                       pltpu.VMEM((NW * 3 * NPW,), jnp.int32),
                       pltpu.VMEM((3 * NPW,), jnp.int32),
                       pltpu.VMEM((NPW,), jnp.float32),
                       pltpu.VMEM((3 * CH,), jnp.int32),
                       pltpu.VMEM((18 * CH,), jnp.float32),
                       pltpu.SemaphoreType.DMA],
    )(lcnt_all, l3_all, dst, *ea_cols, *nf_cols)
    return outs


NDAT = 22


def _tail_body(dat_ref, theta_ref,
               w1et_ref, b1e_ref, w2et_ref, b2e_ref,
               wu1t_ref, bu1_ref, wu2t_ref, bu2_ref,
               gam_ref, bet_ref, wg1t_ref, bg1_ref, wg2t_ref, bg2_ref,
               o_ref, gsum_s, gcnt_s):
    b = pl.program_id(0)

    @pl.when(b == 0)
    def _init():
        gsum_s[...] = jnp.zeros_like(gsum_s)
        gcnt_s[...] = jnp.zeros_like(gcnt_s)

    dat = dat_ref[0]
    cnt = dat[0:1]
    nfT = dat[19:21]
    n1T = jnp.where(cnt > 0.0, dat[1:3], 0.0)
    n2T = jnp.where(cnt > 1.0, dat[3:5], 0.0)
    n3T = jnp.where(cnt > 2.0, dat[5:7], 0.0)

    w1et = w1et_ref[...]; b1e = b1e_ref[...]
    w2et = w2et_ref[...]; b2e = b2e_ref[...]

    def edge_mlp(eaT, j):
        h = jnp.dot(w1et, eaT, preferred_element_type=jnp.float32) + b1e
        h = _leaky(h)
        ef = jnp.tanh(jnp.dot(w2et, h, preferred_element_type=jnp.float32)
                      + b2e) * _PI
        return jnp.where(cnt > j, ef, 0.0)

    ef1 = edge_mlp(dat[7:11], 0.0)
    ef2 = edge_mlp(dat[11:15], 1.0)
    ef3 = edge_mlp(dat[15:19], 2.0)

    cth = jnp.cos(0.5 * jnp.sum(theta_ref[...]))

    def cprod(x):
        c = jnp.cos(0.5 * x)
        return c[0:1] * c[1:2]

    p_nf = cprod(nfT); p_n1 = cprod(n1T)
    pqc = (p_nf * p_nf) * (p_n1 * p_n1) * cprod(n2T) * cprod(n3T)
    pqc = pqc * cprod(ef1) * cprod(ef2) * cprod(ef3) * cth

    deg = jnp.maximum(jnp.minimum(cnt, np.float32(K - 1)), 1.0)
    neigh_mean = (n1T + n2T + n3T) / deg

    wu1t = wu1t_ref[...]
    upre = (jnp.dot(wu1t[:, 0:P], nfT, preferred_element_type=jnp.float32)
            + wu1t[:, P:P + 1] * pqc
            + jnp.dot(wu1t[:, P + 1:2 * P + 1], neigh_mean,
                      preferred_element_type=jnp.float32)
            + bu1_ref[...])
    u = jnp.dot(wu2t_ref[...], _leaky(upre), preferred_element_type=jnp.float32)
    u = u + bu2_ref[...]

    mu = (u[0:1] + u[1:2]) * 0.5
    d = u - mu
    var = (d[0:1] * d[0:1] + d[1:2] * d[1:2]) * 0.5
    un = d / jnp.sqrt(var + 1e-3) * gam_ref[...] + bet_ref[...]

    seg = jax.lax.broadcasted_iota(jnp.int32, (NG, 1), 0).astype(jnp.float32)
    onehot = (dat[21:22] == seg).astype(jnp.float32)
    gsum_s[...] += jax.lax.dot_general(
        un, onehot, (((1,), (1,)), ((), ())), preferred_element_type=jnp.float32)
    gcnt_s[...] += jax.lax.dot_general(
        jnp.ones((1, BN), jnp.float32), onehot, (((1,), (1,)), ((), ())),
        preferred_element_type=jnp.float32)

    @pl.when(b == NB - 1)
    def _fin():
        gmean = gsum_s[...] / jnp.maximum(gcnt_s[...], 1.0)
        g = _leaky(jnp.dot(wg1t_ref[...], gmean,
                           preferred_element_type=jnp.float32) + bg1_ref[...])
        o_ref[...] = (jnp.dot(wg2t_ref[...], g,
                              preferred_element_type=jnp.float32) + bg2_ref[...])


def _tail(dat, theta, W1e, b1e, W2e, b2e, Wu1, bu1, Wu2, bu2, gamma, beta,
          Wg1, bg1, Wg2, bg2):
    full = lambda s: pl.BlockSpec(s, lambda b: (0, 0))
    logits_t = pl.pallas_call(
        _tail_body,
        grid=(NB,),
        in_specs=[pl.BlockSpec((1, NDAT, BN), lambda b: (b, 0, 0)),
                  full((1, 27)),
                  full((H, DE)), full((H, 1)), full((P, H)), full((P, 1)),
                  full((H, 2 * P + 1)), full((H, 1)), full((P, H)), full((P, 1)),
                  full((P, 1)), full((P, 1)),
                  full((NC, P)), full((NC, 1)), full((NC, NC)), full((NC, 1))],
        out_specs=pl.BlockSpec((NC, NG), lambda b: (0, 0)),
        out_shape=jax.ShapeDtypeStruct((NC, NG), jnp.float32),
        scratch_shapes=[pltpu.VMEM((P, NG), jnp.float32),
                        pltpu.VMEM((1, NG), jnp.float32)],
    )(dat, theta.reshape(1, 27),
      W1e.T, b1e.reshape(H, 1), W2e.T, b2e.reshape(P, 1),
      Wu1.T, bu1.reshape(H, 1), Wu2.T, bu2.reshape(P, 1),
      gamma.reshape(P, 1), beta.reshape(P, 1),
      Wg1.T, bg1.reshape(NC, 1), Wg2.T, bg2.reshape(NC, 1))
    return logits_t.T


def kernel(node_feat, edge_attr, edge_index, batch, W1n, b1n, W2n, b2n,
           W1e, b1e, W2e, b2e, theta, Wu1, bu1, Wu2, bu2, gamma, beta,
           Wg1, bg1, Wg2, bg2):
    nf = _node_mlp(node_feat, W1n, b1n, W2n, b2n)

    src = edge_index[0]
    dst = edge_index[1]
    lcnt_all, l3_all = _sc_select(src)
    ea_cols = [edge_attr[:, d] for d in range(DE)]
    nf_cols = [nf[:, c] for c in range(P)]
    outs = _sc_merge(lcnt_all, l3_all, dst, ea_cols, nf_cols)

    rows = list(outs) + nf_cols + [batch.astype(jnp.float32)]
    dat = (jnp.stack(rows, axis=0)
           .reshape(NDAT, NB, BN).transpose(1, 0, 2))
    return _tail(dat, theta, W1e, b1e, W2e, b2e, Wu1, bu1, Wu2, bu2,
                 gamma, beta, Wg1, bg1, Wg2, bg2)
